# baseline (speedup 1.0000x reference)
import functools

import jax
import jax.numpy as jnp
from jax import lax
from jax.experimental import pallas as pl
from jax.experimental.pallas import tpu as pltpu
from jax.experimental.pallas import tpu_sc as plsc

N = 10000
E = 320000
D = 128
NC = 2
NS = 16
NW = NC * NS
EPT = E // NW

CHUNK = 128
BCH = 16
BLK_EDGES = BCH * CHUNK
NBLK0 = 10
NBLK1 = 0
TBLK = NS * (NBLK0 + NBLK1)
E_PAD = TBLK * BLK_EDGES
N_PAD = 10240
ROWS_PT = N_PAD // NS

_mesh = plsc.VectorSubcoreMesh(core_axis_name="c", subcore_axis_name="s")
_sc_params = pltpu.CompilerParams(needs_layout_passes=False)


def _deg_body(dst_hbm, zeros_hbm, degp_hbm, idx_v, deg_v):
    c = lax.axis_index("c")
    s = lax.axis_index("s")
    w = c * NS + s
    pltpu.sync_copy(zeros_hbm, deg_v)
    pltpu.sync_copy(dst_hbm.at[pl.ds(w * EPT, EPT)], idx_v)
    ones = jnp.full((16,), 1.0, dtype=jnp.float32)

    def body(j, carry):
        idx = idx_v[pl.ds(j * 16, 16)]
        plsc.addupdate_scatter(deg_v, [idx], ones)
        return carry

    lax.fori_loop(0, EPT // 16, body, 0)
    pltpu.sync_copy(deg_v, degp_hbm.at[w])


_deg_kernel = functools.partial(
    pl.kernel,
    out_type=jax.ShapeDtypeStruct((NW, N_PAD), jnp.float32),
    mesh=_mesh,
    compiler_params=_sc_params,
    scratch_types=[
        pltpu.VMEM((EPT,), jnp.int32),
        pltpu.VMEM((N_PAD,), jnp.float32),
    ],
)(_deg_body)


def _agg_body(h2_hbm, srcr_hbm, dstr_hbm, zrows_hbm, aggp_hbm,
              src_v, dst_v, rows0, rows1, acc_sh, sem0, sem1):
    c = lax.axis_index("c")
    s = lax.axis_index("s")
    base_blk = jnp.where(c == 0, s * NBLK0, NS * NBLK0 + s * NBLK1)
    nblk = jnp.where(c == 0, NBLK0, NBLK1)
    pltpu.sync_copy(zrows_hbm, acc_sh.at[pl.ds(s * ROWS_PT, ROWS_PT)])
    plsc.subcore_barrier()

    def block(blk, carry):
        pltpu.sync_copy(srcr_hbm.at[base_blk + blk], src_v)
        pltpu.sync_copy(dstr_hbm.at[base_blk + blk], dst_v)
        pltpu.async_copy(h2_hbm.at[src_v.at[0]], rows0, sem0)
        pltpu.async_copy(h2_hbm.at[src_v.at[1]], rows1, sem1)

        def body(i, carry2):
            g = i * 2
            for b, rows, sem in ((0, rows0, sem0), (1, rows1, sem1)):
                j = g + b
                pltpu.make_async_copy(h2_hbm.at[src_v.at[0]], rows, sem).wait()
                pltpu.sync_copy(rows, acc_sh.at[dst_v.at[j]], add=True)
                pltpu.async_copy(h2_hbm.at[src_v.at[j + 2]], rows, sem)
            return carry2

        lax.fori_loop(0, (BCH - 2) // 2, body, 0)
        pltpu.make_async_copy(h2_hbm.at[src_v.at[0]], rows0, sem0).wait()
        pltpu.sync_copy(rows0, acc_sh.at[dst_v.at[BCH - 2]], add=True)
        pltpu.make_async_copy(h2_hbm.at[src_v.at[0]], rows1, sem1).wait()
        pltpu.sync_copy(rows1, acc_sh.at[dst_v.at[BCH - 1]], add=True)
        return carry

    lax.fori_loop(0, nblk, block, 0)
    plsc.subcore_barrier()
    pltpu.sync_copy(acc_sh.at[pl.ds(s * ROWS_PT, ROWS_PT)],
                    aggp_hbm.at[c, pl.ds(s * ROWS_PT, ROWS_PT)])


_agg_kernel = functools.partial(
    pl.kernel,
    out_type=jax.ShapeDtypeStruct((NC, N_PAD, D), jnp.float32),
    mesh=_mesh,
    compiler_params=_sc_params,
    scratch_types=[
        pltpu.VMEM((BCH, CHUNK), jnp.int32),
        pltpu.VMEM((BCH, CHUNK), jnp.int32),
        pltpu.VMEM((CHUNK, D), jnp.float32),
        pltpu.VMEM((CHUNK, D), jnp.float32),
        pltpu.VMEM_SHARED((N_PAD, D), jnp.float32),
        pltpu.SemaphoreType.DMA,
        pltpu.SemaphoreType.DMA,
    ],
)(_agg_body)


_RBH = 2048


def _h2_body(x_ref, w_ref, degp_ref, h2_ref, dis_ref):
    i = pl.program_id(0)
    deg = jnp.sum(degp_ref[:, pl.ds(i * _RBH, _RBH)], axis=0) + 1.0
    dis = lax.rsqrt(deg)
    h = jnp.dot(x_ref[...], w_ref[...], preferred_element_type=jnp.float32)
    h2_ref[...] = h * dis[:, None]
    dis_ref[...] = jnp.broadcast_to(dis[:, None], (_RBH, 8))


def _h2_call(x_pad, W, degp):
    return pl.pallas_call(
        _h2_body,
        grid=(N_PAD // _RBH,),
        in_specs=[
            pl.BlockSpec((_RBH, D), lambda i: (i, 0)),
            pl.BlockSpec((D, D), lambda i: (0, 0)),
            pl.BlockSpec((NW, N_PAD), lambda i: (0, 0)),
        ],
        out_specs=[
            pl.BlockSpec((_RBH, D), lambda i: (i, 0)),
            pl.BlockSpec((_RBH, 8), lambda i: (i, 0)),
        ],
        out_shape=[
            jax.ShapeDtypeStruct((N_PAD, D), jnp.float32),
            jax.ShapeDtypeStruct((N_PAD, 8), jnp.float32),
        ],
    )(x_pad, W, degp)


_RBO = 2000


def _out_body(agg0_ref, agg1_ref, h2_ref, dis_ref, b_ref, out_ref):
    dis = dis_ref[...][:, 0]
    tot = (agg0_ref[0] + agg1_ref[0] + h2_ref[...]) * dis[:, None]
    out_ref[...] = jnp.maximum(tot + b_ref[...], 0.0)


def _out_call(aggp, h2, disb, b2d):
    nb = N // _RBO
    return pl.pallas_call(
        _out_body,
        grid=(nb,),
        in_specs=[
            pl.BlockSpec((1, _RBO, D), lambda i: (0, i, 0)),
            pl.BlockSpec((1, _RBO, D), lambda i: (1, i, 0)),
            pl.BlockSpec((_RBO, D), lambda i: (i, 0)),
            pl.BlockSpec((_RBO, 8), lambda i: (i, 0)),
            pl.BlockSpec((1, D), lambda i: (0, 0)),
        ],
        out_specs=pl.BlockSpec((_RBO, D), lambda i: (i, 0)),
        out_shape=jax.ShapeDtypeStruct((N, D), jnp.float32),
    )(aggp, aggp, h2, disb, b2d)


def kernel(x, edge_index, W, b):
    src = edge_index[0]
    dst = edge_index[1]
    npad = E_PAD - E
    src_pad = jnp.concatenate([src, jnp.full((npad,), N, jnp.int32)])
    pad_dst = jnp.arange(npad, dtype=jnp.int32) % N_PAD
    dst_pad = jnp.concatenate([dst, pad_dst])
    srcr = src_pad.reshape(TBLK, BCH, CHUNK)
    dstr = dst_pad.reshape(TBLK, BCH, CHUNK)
    x_pad = jnp.concatenate([x, jnp.zeros((N_PAD - N, D), jnp.float32)])
    zeros1 = jnp.zeros((N_PAD,), jnp.float32)
    zrows = jnp.zeros((ROWS_PT, D), jnp.float32)

    degp = _deg_kernel(dst, zeros1)
    h2, disb = _h2_call(x_pad, W, degp)
    aggp = _agg_kernel(h2, srcr, dstr, zrows)
    return _out_call(aggp, h2, disb, b.reshape(1, D))


# speedup vs baseline: 1.1130x; 1.1130x over previous
import functools

import jax
import jax.numpy as jnp
from jax import lax
from jax.experimental import pallas as pl
from jax.experimental.pallas import tpu as pltpu
from jax.experimental.pallas import tpu_sc as plsc

N = 10000
E = 320000
D = 128
NC = 2
NS = 16
NW = NC * NS
EPT = E // NW

CHUNK = 128
BCH = 16
BLK_EDGES = BCH * CHUNK
NBLK0 = 5
NBLK1 = 5
TBLK = NS * (NBLK0 + NBLK1)
E_PAD = TBLK * BLK_EDGES
N_PAD = 10240
ROWS_PT = N_PAD // NS

_mesh = plsc.VectorSubcoreMesh(core_axis_name="c", subcore_axis_name="s")
_sc_params = pltpu.CompilerParams(needs_layout_passes=False)


def _deg_body(dst_hbm, zeros_hbm, degp_hbm, idx_v, deg_v):
    c = lax.axis_index("c")
    s = lax.axis_index("s")
    w = c * NS + s
    pltpu.sync_copy(zeros_hbm, deg_v)
    pltpu.sync_copy(dst_hbm.at[pl.ds(w * EPT, EPT)], idx_v)
    ones = jnp.full((16,), 1.0, dtype=jnp.float32)

    def body(j, carry):
        idx = idx_v[pl.ds(j * 16, 16)]
        plsc.addupdate_scatter(deg_v, [idx], ones)
        return carry

    lax.fori_loop(0, EPT // 16, body, 0)
    pltpu.sync_copy(deg_v, degp_hbm.at[w])


_deg_kernel = functools.partial(
    pl.kernel,
    out_type=jax.ShapeDtypeStruct((NW, N_PAD), jnp.float32),
    mesh=_mesh,
    compiler_params=_sc_params,
    scratch_types=[
        pltpu.VMEM((EPT,), jnp.int32),
        pltpu.VMEM((N_PAD,), jnp.float32),
    ],
)(_deg_body)


def _agg_body(h2_hbm, srcr_hbm, dstr_hbm, zrows_hbm, aggp_hbm,
              src_v, dst_v, rows0, rows1, acc_sh, sem0, sem1):
    c = lax.axis_index("c")
    s = lax.axis_index("s")
    base_blk = jnp.where(c == 0, s * NBLK0, NS * NBLK0 + s * NBLK1)
    nblk = jnp.where(c == 0, NBLK0, NBLK1)
    pltpu.sync_copy(zrows_hbm, acc_sh.at[pl.ds(s * ROWS_PT, ROWS_PT)])
    plsc.subcore_barrier()

    def block(blk, carry):
        pltpu.sync_copy(srcr_hbm.at[base_blk + blk], src_v)
        pltpu.sync_copy(dstr_hbm.at[base_blk + blk], dst_v)
        pltpu.async_copy(h2_hbm.at[src_v.at[0]], rows0, sem0)
        pltpu.async_copy(h2_hbm.at[src_v.at[1]], rows1, sem1)

        def body(i, carry2):
            g = i * 2
            for b, rows, sem in ((0, rows0, sem0), (1, rows1, sem1)):
                j = g + b
                pltpu.make_async_copy(h2_hbm.at[src_v.at[0]], rows, sem).wait()
                pltpu.sync_copy(rows, acc_sh.at[dst_v.at[j]], add=True)
                pltpu.async_copy(h2_hbm.at[src_v.at[j + 2]], rows, sem)
            return carry2

        lax.fori_loop(0, (BCH - 2) // 2, body, 0)
        pltpu.make_async_copy(h2_hbm.at[src_v.at[0]], rows0, sem0).wait()
        pltpu.sync_copy(rows0, acc_sh.at[dst_v.at[BCH - 2]], add=True)
        pltpu.make_async_copy(h2_hbm.at[src_v.at[0]], rows1, sem1).wait()
        pltpu.sync_copy(rows1, acc_sh.at[dst_v.at[BCH - 1]], add=True)
        return carry

    lax.fori_loop(0, nblk, block, 0)
    plsc.subcore_barrier()
    pltpu.sync_copy(acc_sh.at[pl.ds(s * ROWS_PT, ROWS_PT)],
                    aggp_hbm.at[c, pl.ds(s * ROWS_PT, ROWS_PT)])


_agg_kernel = functools.partial(
    pl.kernel,
    out_type=jax.ShapeDtypeStruct((NC, N_PAD, D), jnp.float32),
    mesh=_mesh,
    compiler_params=_sc_params,
    scratch_types=[
        pltpu.VMEM((BCH, CHUNK), jnp.int32),
        pltpu.VMEM((BCH, CHUNK), jnp.int32),
        pltpu.VMEM((CHUNK, D), jnp.float32),
        pltpu.VMEM((CHUNK, D), jnp.float32),
        pltpu.VMEM_SHARED((N_PAD, D), jnp.float32),
        pltpu.SemaphoreType.DMA,
        pltpu.SemaphoreType.DMA,
    ],
)(_agg_body)


_RBH = 2048


def _h2_body(x_ref, w_ref, degp_ref, h2_ref, dis_ref):
    i = pl.program_id(0)
    deg = jnp.sum(degp_ref[:, pl.ds(i * _RBH, _RBH)], axis=0) + 1.0
    dis = lax.rsqrt(deg)
    h = jnp.dot(x_ref[...], w_ref[...], preferred_element_type=jnp.float32)
    h2_ref[...] = h * dis[:, None]
    dis_ref[...] = jnp.broadcast_to(dis[:, None], (_RBH, 8))


def _h2_call(x_pad, W, degp):
    return pl.pallas_call(
        _h2_body,
        grid=(N_PAD // _RBH,),
        in_specs=[
            pl.BlockSpec((_RBH, D), lambda i: (i, 0)),
            pl.BlockSpec((D, D), lambda i: (0, 0)),
            pl.BlockSpec((NW, N_PAD), lambda i: (0, 0)),
        ],
        out_specs=[
            pl.BlockSpec((_RBH, D), lambda i: (i, 0)),
            pl.BlockSpec((_RBH, 8), lambda i: (i, 0)),
        ],
        out_shape=[
            jax.ShapeDtypeStruct((N_PAD, D), jnp.float32),
            jax.ShapeDtypeStruct((N_PAD, 8), jnp.float32),
        ],
    )(x_pad, W, degp)


_RBO = 2000


def _out_body(agg0_ref, agg1_ref, h2_ref, dis_ref, b_ref, out_ref):
    dis = dis_ref[...][:, 0]
    tot = (agg0_ref[0] + agg1_ref[0] + h2_ref[...]) * dis[:, None]
    out_ref[...] = jnp.maximum(tot + b_ref[...], 0.0)


def _out_call(aggp, h2, disb, b2d):
    nb = N // _RBO
    return pl.pallas_call(
        _out_body,
        grid=(nb,),
        in_specs=[
            pl.BlockSpec((1, _RBO, D), lambda i: (0, i, 0)),
            pl.BlockSpec((1, _RBO, D), lambda i: (1, i, 0)),
            pl.BlockSpec((_RBO, D), lambda i: (i, 0)),
            pl.BlockSpec((_RBO, 8), lambda i: (i, 0)),
            pl.BlockSpec((1, D), lambda i: (0, 0)),
        ],
        out_specs=pl.BlockSpec((_RBO, D), lambda i: (i, 0)),
        out_shape=jax.ShapeDtypeStruct((N, D), jnp.float32),
    )(aggp, aggp, h2, disb, b2d)


def kernel(x, edge_index, W, b):
    src = edge_index[0]
    dst = edge_index[1]
    npad = E_PAD - E
    src_pad = jnp.concatenate([src, jnp.full((npad,), N, jnp.int32)])
    pad_dst = jnp.arange(npad, dtype=jnp.int32) % N_PAD
    dst_pad = jnp.concatenate([dst, pad_dst])
    srcr = src_pad.reshape(TBLK, BCH, CHUNK)
    dstr = dst_pad.reshape(TBLK, BCH, CHUNK)
    x_pad = jnp.concatenate([x, jnp.zeros((N_PAD - N, D), jnp.float32)])
    zeros1 = jnp.zeros((N_PAD,), jnp.float32)
    zrows = jnp.zeros((ROWS_PT, D), jnp.float32)

    degp = _deg_kernel(dst, zeros1)
    h2, disb = _h2_call(x_pad, W, degp)
    aggp = _agg_kernel(h2, srcr, dstr, zrows)
    return _out_call(aggp, h2, disb, b.reshape(1, D))
